# baseline (speedup 1.0000x reference)
import jax
import jax.numpy as jnp
from jax import lax
from jax.experimental import pallas as pl
from jax.experimental.pallas import tpu as pltpu
from jax.experimental.pallas import tpu_sc as plsc

R = 128
C = 100000
BLK = 4096
NBLK = -(-C // BLK)

NC = 2
NS = 16
NW = NC * NS
RPW = R // NW
WIN = 16


def _scan_body(x_ref, idx_ref, rmax_ref, ridx_ref):
    k = pl.program_id(0)

    @pl.when(k == 0)
    def _init():
        rmax_ref[...] = jnp.full((R, 1), -jnp.inf, jnp.float32)
        ridx_ref[...] = jnp.zeros((R, 1), jnp.int32)

    cols = lax.broadcasted_iota(jnp.int32, (R, BLK), 1) + k * BLK
    v = jnp.where(cols < C, x_ref[...], -jnp.inf)
    m = jnp.max(v, axis=1, keepdims=True)
    loc = jnp.min(jnp.where(v == m, cols, jnp.int32(2**30)),
                  axis=1, keepdims=True)
    upd = m > rmax_ref[...]
    rmax_ref[...] = jnp.where(upd, m, rmax_ref[...])
    ridx_ref[...] = jnp.where(upd, loc, ridx_ref[...])

    @pl.when(k == NBLK - 1)
    def _fin():
        idx_ref[...] = ridx_ref[...]


def _argmax_tc(x):
    return pl.pallas_call(
        _scan_body,
        grid=(NBLK,),
        in_specs=[pl.BlockSpec((R, BLK), lambda k: (0, k))],
        out_specs=pl.BlockSpec((R, 1), lambda k: (0, 0)),
        out_shape=jax.ShapeDtypeStruct((R, 1), jnp.int32),
        scratch_shapes=[
            pltpu.VMEM((R, 1), jnp.float32),
            pltpu.VMEM((R, 1), jnp.int32),
        ],
    )(x)


def _sc_body(x_hbm, idx_hbm, out_hbm, idx_v, win_v, out_v, sem):
    c = lax.axis_index("c")
    s = lax.axis_index("s")
    w = s * NC + c

    pltpu.sync_copy(idx_hbm.at[w // RPW], idx_v)
    lane = lax.iota(jnp.int32, 16)
    base = (w % RPW) * RPW
    iv = idx_v[...]

    def take(vec, ids):
        dnums = lax.GatherDimensionNumbers(
            offset_dims=(), collapsed_slice_dims=(0,), start_index_map=(0,))
        return lax.gather(vec, ids[:, None], dnums, (1,),
                          mode=lax.GatherScatterMode.PROMISE_IN_BOUNDS)

    k2 = (lane >> 1) & 3
    idx2 = take(iv, base + k2)
    p2 = (RPW * w + k2) * C + idx2
    q0 = jnp.maximum(p2 - 1, 0) >> 7
    qvec = jnp.minimum(q0 + (lane & 1), R * C // 128 - 1)
    pltpu.async_copy(x_hbm.at[qvec], win_v, sem).wait()

    acc = jnp.zeros((16,), jnp.float32)
    for k in range(RPW):
        idxk = take(iv, jnp.full((16,), base + k, jnp.int32))
        pk = (RPW * w + k) * C + idxk
        q0k = jnp.maximum(pk - 1, 0) >> 7
        blocks = [[win_v[2 * k + r, 16 * s:16 * (s + 1)] for s in range(8)]
                  for r in range(2)]

        def pick(delta, _pk=pk, _q0k=q0k, _blocks=blocks):
            loc = jnp.clip(_pk + delta - (_q0k << 7), 0, 255)
            sub = loc >> 4
            sel = jnp.zeros((16,), jnp.float32)
            for r in range(2):
                for s in range(8):
                    m = (1 - jnp.minimum(jnp.abs(sub - (8 * r + s)), 1))
                    sel = sel + _blocks[r][s] * m.astype(jnp.float32)
            return take(sel, loc & 15)

        center = pick(0)
        left = pick(-1) * jnp.minimum(idxk, 1).astype(jnp.float32)
        right = pick(1) * jnp.minimum(C - 1 - idxk, 1).astype(jnp.float32)
        conf = center + jnp.maximum(left, right)
        acc = jnp.where(lane == k, conf, acc)

    out_v[...] = acc
    pltpu.sync_copy(out_v, out_hbm.at[w])


def _confidence_sc(x2d, idx2d):
    mesh = plsc.VectorSubcoreMesh(core_axis_name="c", subcore_axis_name="s")
    return pl.kernel(
        _sc_body,
        out_type=jax.ShapeDtypeStruct((NW, 16), jnp.float32),
        mesh=mesh,
        scratch_types=[
            pltpu.VMEM((16,), jnp.int32),
            pltpu.VMEM((16, 128), jnp.float32),
            pltpu.VMEM((16,), jnp.float32),
            pltpu.SemaphoreType.DMA,
        ],
    )(x2d, idx2d)


def kernel(tensor_smax):
    idx = _argmax_tc(tensor_smax)
    out2d = _confidence_sc(tensor_smax.reshape(R * C // 128, 128),
                           idx.reshape(8, 16))
    return out2d[:, :RPW].reshape(-1)


# speedup vs baseline: 1.2943x; 1.2943x over previous
import jax
import jax.numpy as jnp
from jax import lax
from jax.experimental import pallas as pl
from jax.experimental.pallas import tpu as pltpu
from jax.experimental.pallas import tpu_sc as plsc

R = 128
C = 100000
BLK = 4096
NBLK = -(-C // BLK)
NCH = BLK // 128

NC = 2
NS = 16
NW = NC * NS
RPW = R // NW


def _scan_body(x_ref, maxo_ref, idxo_ref, wao_ref, wbo_ref,
               rmax_ref, ridx_ref, wa_ref, wb_ref, carry_ref):
    k = pl.program_id(0)

    @pl.when(k == 0)
    def _init():
        rmax_ref[...] = jnp.full((R, 1), -jnp.inf, jnp.float32)
        ridx_ref[...] = jnp.zeros((R, 1), jnp.int32)
        wa_ref[...] = jnp.zeros((R, 128), jnp.float32)
        wb_ref[...] = jnp.zeros((R, 128), jnp.float32)
        carry_ref[...] = jnp.zeros((R, 128), jnp.float32)

    cols = lax.broadcasted_iota(jnp.int32, (R, BLK), 1) + k * BLK
    x = x_ref[...]
    valid = cols < C
    vm = jnp.where(valid, x, -jnp.inf)
    v0 = jnp.where(valid, x, 0.0)
    v3 = v0.reshape(R, NCH, 128)

    @pl.when(k > 0)
    def _pending():
        fill = ridx_ref[...] == k * BLK - 1
        wb_ref[...] = jnp.where(fill, v3[:, 0, :], wb_ref[...])

    m = jnp.max(vm, axis=1, keepdims=True)
    loc = jnp.min(jnp.where(vm == m, cols, jnp.int32(2**30)),
                  axis=1, keepdims=True)

    cid = lax.broadcasted_iota(jnp.int32, (R, NCH, 1), 1) + k * NCH
    a = (jnp.maximum(loc - 1, 0) >> 7).reshape(R, 1, 1)
    b = ((loc + 1) >> 7).reshape(R, 1, 1)
    wa_new = jnp.sum(jnp.where(cid == a, v3, 0.0), axis=1)
    wb_new = jnp.sum(jnp.where(cid == b, v3, 0.0), axis=1)
    wa_new = jnp.where(a.reshape(R, 1) < k * NCH, carry_ref[...], wa_new)

    upd = m > rmax_ref[...]
    rmax_ref[...] = jnp.where(upd, m, rmax_ref[...])
    ridx_ref[...] = jnp.where(upd, loc, ridx_ref[...])
    wa_ref[...] = jnp.where(upd, wa_new, wa_ref[...])
    wb_ref[...] = jnp.where(upd, wb_new, wb_ref[...])
    carry_ref[...] = v3[:, NCH - 1, :]

    @pl.when(k == NBLK - 1)
    def _fin():
        maxo_ref[...] = rmax_ref[...]
        idxo_ref[...] = ridx_ref[...]
        wao_ref[...] = wa_ref[...]
        wbo_ref[...] = wb_ref[...]


def _argmax_tc(x):
    return pl.pallas_call(
        _scan_body,
        grid=(NBLK,),
        in_specs=[pl.BlockSpec((R, BLK), lambda k: (0, k))],
        out_specs=[
            pl.BlockSpec((R, 1), lambda k: (0, 0)),
            pl.BlockSpec((R, 1), lambda k: (0, 0)),
            pl.BlockSpec((R, 128), lambda k: (0, 0)),
            pl.BlockSpec((R, 128), lambda k: (0, 0)),
        ],
        out_shape=[
            jax.ShapeDtypeStruct((R, 1), jnp.float32),
            jax.ShapeDtypeStruct((R, 1), jnp.int32),
            jax.ShapeDtypeStruct((R, 128), jnp.float32),
            jax.ShapeDtypeStruct((R, 128), jnp.float32),
        ],
        scratch_shapes=[
            pltpu.VMEM((R, 1), jnp.float32),
            pltpu.VMEM((R, 1), jnp.int32),
            pltpu.VMEM((R, 128), jnp.float32),
            pltpu.VMEM((R, 128), jnp.float32),
            pltpu.VMEM((R, 128), jnp.float32),
        ],
    )(x)


def _sc_body(wa_hbm, wb_hbm, idx_hbm, max_hbm, out_hbm,
             idx_v, max_v, wa_v, wb_v, out_v, sem_a, sem_b):
    c = lax.axis_index("c")
    s = lax.axis_index("s")
    w = s * NC + c

    pltpu.sync_copy(idx_hbm.at[w // RPW], idx_v)
    pltpu.sync_copy(max_hbm.at[w // RPW], max_v)
    lane = lax.iota(jnp.int32, 16)
    base = (w % RPW) * RPW
    iv = idx_v[...]
    mv = max_v[...]

    def take(vec, ids):
        dnums = lax.GatherDimensionNumbers(
            offset_dims=(), collapsed_slice_dims=(0,), start_index_map=(0,))
        return lax.gather(vec, ids[:, None], dnums, (1,),
                          mode=lax.GatherScatterMode.PROMISE_IN_BOUNDS)

    qv = RPW * w + (lane & 3)
    cp_a = pltpu.async_copy(wa_hbm.at[qv], wa_v, sem_a)
    cp_b = pltpu.async_copy(wb_hbm.at[qv], wb_v, sem_b)
    cp_a.wait()
    cp_b.wait()

    def treepick(win_ref, k, pos):
        sub = pos >> 4
        sel = jnp.zeros((16,), jnp.float32)
        for t in range(8):
            mt = (1 - jnp.minimum(jnp.abs(sub - t), 1)).astype(jnp.float32)
            sel = sel + win_ref[k, 16 * t:16 * (t + 1)] * mt
        return take(sel, pos & 15)

    acc = jnp.zeros((16,), jnp.float32)
    for k in range(RPW):
        sel_k = jnp.full((16,), base + k, jnp.int32)
        idxk = take(iv, sel_k)
        maxk = take(mv, sel_k)
        left = treepick(wa_v, k, (idxk - 1) & 127)
        left = left * jnp.minimum(idxk, 1).astype(jnp.float32)
        right = treepick(wb_v, k, (idxk + 1) & 127)
        right = right * jnp.minimum(C - 1 - idxk, 1).astype(jnp.float32)
        conf = maxk + jnp.maximum(left, right)
        acc = jnp.where(lane == k, conf, acc)

    out_v[...] = acc
    pltpu.sync_copy(out_v, out_hbm.at[w])


def _confidence_sc(wa, wb, idx2d, max2d):
    mesh = plsc.VectorSubcoreMesh(core_axis_name="c", subcore_axis_name="s")
    return pl.kernel(
        _sc_body,
        out_type=jax.ShapeDtypeStruct((NW, 16), jnp.float32),
        mesh=mesh,
        scratch_types=[
            pltpu.VMEM((16,), jnp.int32),
            pltpu.VMEM((16,), jnp.float32),
            pltpu.VMEM((16, 128), jnp.float32),
            pltpu.VMEM((16, 128), jnp.float32),
            pltpu.VMEM((16,), jnp.float32),
            pltpu.SemaphoreType.DMA,
            pltpu.SemaphoreType.DMA,
        ],
    )(wa, wb, idx2d, max2d)


def kernel(tensor_smax):
    maxv, idx, wa, wb = _argmax_tc(tensor_smax)
    out2d = _confidence_sc(wa, wb, idx.reshape(8, 16), maxv.reshape(8, 16))
    return out2d[:, :RPW].reshape(-1)


# speedup vs baseline: 1.5662x; 1.2100x over previous
import jax
import jax.numpy as jnp
from jax import lax
from jax.experimental import pallas as pl
from jax.experimental.pallas import tpu as pltpu
from jax.experimental.pallas import tpu_sc as plsc

R = 128
C = 100000
BLK = 4096
NBLK = -(-C // BLK)
NCH = BLK // 128

NC = 2
NS = 16
NW = NC * NS
RPW = R // NW


def _scan_body(x_ref, maxo_ref, idxo_ref, wao_ref, wbo_ref,
               rmax_ref, ridx_ref, wa_ref, wb_ref, carry_ref):
    k = pl.program_id(0)

    @pl.when(k == 0)
    def _init():
        rmax_ref[...] = jnp.full((R, 1), -jnp.inf, jnp.float32)
        ridx_ref[...] = jnp.zeros((R, 1), jnp.int32)
        wa_ref[...] = jnp.zeros((R, 128), jnp.float32)
        wb_ref[...] = jnp.zeros((R, 128), jnp.float32)
        carry_ref[...] = jnp.zeros((R, 128), jnp.float32)

    cols = lax.broadcasted_iota(jnp.int32, (R, BLK), 1) + k * BLK
    x = x_ref[...]
    valid = cols < C
    vm = jnp.where(valid, x, -jnp.inf)
    v0 = jnp.where(valid, x, 0.0)

    @pl.when(k > 0)
    def _pending():
        fill = ridx_ref[...] == k * BLK - 1
        wb_ref[...] = jnp.where(fill, v0[:, :128], wb_ref[...])

    m = jnp.max(vm, axis=1, keepdims=True)
    loc = jnp.min(jnp.where(vm == m, cols, jnp.int32(2**30)),
                  axis=1, keepdims=True)

    a = jnp.maximum(loc - 1, 0) >> 7
    b = (loc + 1) >> 7
    wa_new = jnp.zeros((R, 128), jnp.float32)
    wb_new = jnp.zeros((R, 128), jnp.float32)
    for j in range(NCH):
        cj = v0[:, 128 * j:128 * (j + 1)]
        gj = k * NCH + j
        wa_new = jnp.where(a == gj, cj, wa_new)
        wb_new = jnp.where(b == gj, cj, wb_new)
    wa_new = jnp.where(a < k * NCH, carry_ref[...], wa_new)

    upd = m > rmax_ref[...]
    rmax_ref[...] = jnp.where(upd, m, rmax_ref[...])
    ridx_ref[...] = jnp.where(upd, loc, ridx_ref[...])
    wa_ref[...] = jnp.where(upd, wa_new, wa_ref[...])
    wb_ref[...] = jnp.where(upd, wb_new, wb_ref[...])
    carry_ref[...] = v0[:, BLK - 128:]

    @pl.when(k == NBLK - 1)
    def _fin():
        maxo_ref[...] = rmax_ref[...]
        idxo_ref[...] = ridx_ref[...]
        wao_ref[...] = wa_ref[...]
        wbo_ref[...] = wb_ref[...]


def _argmax_tc(x):
    return pl.pallas_call(
        _scan_body,
        grid=(NBLK,),
        in_specs=[pl.BlockSpec((R, BLK), lambda k: (0, k))],
        out_specs=[
            pl.BlockSpec((R, 1), lambda k: (0, 0)),
            pl.BlockSpec((R, 1), lambda k: (0, 0)),
            pl.BlockSpec((R, 128), lambda k: (0, 0)),
            pl.BlockSpec((R, 128), lambda k: (0, 0)),
        ],
        out_shape=[
            jax.ShapeDtypeStruct((R, 1), jnp.float32),
            jax.ShapeDtypeStruct((R, 1), jnp.int32),
            jax.ShapeDtypeStruct((R, 128), jnp.float32),
            jax.ShapeDtypeStruct((R, 128), jnp.float32),
        ],
        scratch_shapes=[
            pltpu.VMEM((R, 1), jnp.float32),
            pltpu.VMEM((R, 1), jnp.int32),
            pltpu.VMEM((R, 128), jnp.float32),
            pltpu.VMEM((R, 128), jnp.float32),
            pltpu.VMEM((R, 128), jnp.float32),
        ],
    )(x)


def _sc_body(wa_hbm, wb_hbm, idx_hbm, max_hbm, out_hbm,
             idx_v, max_v, wa_v, wb_v, out_v, sem_a, sem_b):
    c = lax.axis_index("c")
    s = lax.axis_index("s")
    w = s * NC + c

    pltpu.sync_copy(idx_hbm.at[w // RPW], idx_v)
    pltpu.sync_copy(max_hbm.at[w // RPW], max_v)
    lane = lax.iota(jnp.int32, 16)
    base = (w % RPW) * RPW
    iv = idx_v[...]
    mv = max_v[...]

    def take(vec, ids):
        dnums = lax.GatherDimensionNumbers(
            offset_dims=(), collapsed_slice_dims=(0,), start_index_map=(0,))
        return lax.gather(vec, ids[:, None], dnums, (1,),
                          mode=lax.GatherScatterMode.PROMISE_IN_BOUNDS)

    qv = RPW * w + (lane & 3)
    cp_a = pltpu.async_copy(wa_hbm.at[qv], wa_v, sem_a)
    cp_b = pltpu.async_copy(wb_hbm.at[qv], wb_v, sem_b)
    cp_a.wait()
    cp_b.wait()

    def treepick(win_ref, k, pos):
        sub = pos >> 4
        sel = jnp.zeros((16,), jnp.float32)
        for t in range(8):
            mt = (1 - jnp.minimum(jnp.abs(sub - t), 1)).astype(jnp.float32)
            sel = sel + win_ref[k, 16 * t:16 * (t + 1)] * mt
        return take(sel, pos & 15)

    acc = jnp.zeros((16,), jnp.float32)
    for k in range(RPW):
        sel_k = jnp.full((16,), base + k, jnp.int32)
        idxk = take(iv, sel_k)
        maxk = take(mv, sel_k)
        left = treepick(wa_v, k, (idxk - 1) & 127)
        left = left * jnp.minimum(idxk, 1).astype(jnp.float32)
        right = treepick(wb_v, k, (idxk + 1) & 127)
        right = right * jnp.minimum(C - 1 - idxk, 1).astype(jnp.float32)
        conf = maxk + jnp.maximum(left, right)
        acc = jnp.where(lane == k, conf, acc)

    out_v[...] = acc
    pltpu.sync_copy(out_v, out_hbm.at[w])


def _confidence_sc(wa, wb, idx2d, max2d):
    mesh = plsc.VectorSubcoreMesh(core_axis_name="c", subcore_axis_name="s")
    return pl.kernel(
        _sc_body,
        out_type=jax.ShapeDtypeStruct((NW, 16), jnp.float32),
        mesh=mesh,
        scratch_types=[
            pltpu.VMEM((16,), jnp.int32),
            pltpu.VMEM((16,), jnp.float32),
            pltpu.VMEM((16, 128), jnp.float32),
            pltpu.VMEM((16, 128), jnp.float32),
            pltpu.VMEM((16,), jnp.float32),
            pltpu.SemaphoreType.DMA,
            pltpu.SemaphoreType.DMA,
        ],
    )(wa, wb, idx2d, max2d)


def kernel(tensor_smax):
    maxv, idx, wa, wb = _argmax_tc(tensor_smax)
    out2d = _confidence_sc(wa, wb, idx.reshape(8, 16), maxv.reshape(8, 16))
    return out2d[:, :RPW].reshape(-1)


# speedup vs baseline: 2.2202x; 1.4176x over previous
import jax
import jax.numpy as jnp
from jax import lax
from jax.experimental import pallas as pl
from jax.experimental.pallas import tpu as pltpu
from jax.experimental.pallas import tpu_sc as plsc

R = 128
C = 100000
BLKR = 4000
NBLK = C // BLKR

NC = 2
NS = 16
NW = NC * NS
RPW = R // NW


def _shard_body(x_ref, m_ref, loc_ref, l_ref, r_ref, first_ref, last_ref):
    k = pl.program_id(0)
    v = x_ref[...]
    rows = lax.broadcasted_iota(jnp.int32, (BLKR, R), 0) + k * BLKR
    m = jnp.max(v, axis=0, keepdims=True)
    loc = jnp.min(jnp.where(v == m, rows, jnp.int32(2**30)),
                  axis=0, keepdims=True)
    left = jnp.sum(jnp.where(rows == loc - 1, v, 0.0), axis=0, keepdims=True)
    right = jnp.sum(jnp.where(rows == loc + 1, v, 0.0), axis=0, keepdims=True)
    m_ref[...] = m.reshape(1, 1, R)
    loc_ref[...] = loc.reshape(1, 1, R)
    l_ref[...] = left.reshape(1, 1, R)
    r_ref[...] = right.reshape(1, 1, R)
    first_ref[...] = v[0:1, :].reshape(1, 1, R)
    last_ref[...] = v[BLKR - 1:BLKR, :].reshape(1, 1, R)


def _shard_scan_tc(xt):
    o = pl.BlockSpec((1, 1, R), lambda k: (k, 0, 0))
    sf = jax.ShapeDtypeStruct((NBLK, 1, R), jnp.float32)
    si = jax.ShapeDtypeStruct((NBLK, 1, R), jnp.int32)
    return pl.pallas_call(
        _shard_body,
        grid=(NBLK,),
        in_specs=[pl.BlockSpec((BLKR, R), lambda k: (k, 0))],
        out_specs=[o, o, o, o, o, o],
        out_shape=[sf, si, sf, sf, sf, sf],
    )(xt)


def _sc_body(m_hbm, loc_hbm, l_hbm, r_hbm, first_hbm, last_hbm, out_hbm,
             m_v, loc_v, l_v, r_v, first_v, last_v, out_v):
    c = lax.axis_index("c")
    s = lax.axis_index("s")
    w = s * NC + c
    t = w // RPW
    lane = lax.iota(jnp.int32, 16)

    pltpu.sync_copy(m_hbm.at[t], m_v)
    pltpu.sync_copy(loc_hbm.at[t], loc_v)
    pltpu.sync_copy(l_hbm.at[t], l_v)
    pltpu.sync_copy(r_hbm.at[t], r_v)
    pltpu.sync_copy(first_hbm.at[t], first_v)
    pltpu.sync_copy(last_hbm.at[t], last_v)

    def f32(x):
        return x.astype(jnp.float32)

    def eqi(x, const):
        return 1 - jnp.minimum(jnp.abs(x - const), 1)

    M = m_v[0]
    LOC = loc_v[0]
    L = l_v[0]
    RB = r_v[0]
    for sh in range(1, NBLK):
        m_s = m_v[sh]
        loc_s = loc_v[sh]
        fr = f32(eqi(LOC, sh * BLKR - 1))
        RB = first_v[sh] * fr + RB * (1.0 - fr)
        fl = f32(eqi(loc_s, sh * BLKR))
        l_s = last_v[sh - 1] * fl + l_v[sh] * (1.0 - fl)
        uf = jnp.maximum(jnp.sign(m_s - M), 0.0)
        ui = uf.astype(jnp.int32)
        M = m_s * uf + M * (1.0 - uf)
        LOC = loc_s * ui + LOC * (1 - ui)
        L = l_s * uf + L * (1.0 - uf)
        RB = r_v[sh] * uf + RB * (1.0 - uf)

    conf = M + jnp.maximum(L, RB)

    def take(vec, ids):
        dnums = lax.GatherDimensionNumbers(
            offset_dims=(), collapsed_slice_dims=(0,), start_index_map=(0,))
        return lax.gather(vec, ids[:, None], dnums, (1,),
                          mode=lax.GatherScatterMode.PROMISE_IN_BOUNDS)

    out_v[...] = take(conf, 4 * (w % RPW) + (lane & 3))
    pltpu.sync_copy(out_v, out_hbm.at[w])


def _merge_sc(m, loc, l, r, first, last):
    mesh = plsc.VectorSubcoreMesh(core_axis_name="c", subcore_axis_name="s")
    vf = pltpu.VMEM((NBLK, 16), jnp.float32)
    return pl.kernel(
        _sc_body,
        out_type=jax.ShapeDtypeStruct((NW, 16), jnp.float32),
        mesh=mesh,
        scratch_types=[
            vf, pltpu.VMEM((NBLK, 16), jnp.int32), vf, vf, vf, vf,
            pltpu.VMEM((16,), jnp.float32),
        ],
    )(m, loc, l, r, first, last)


def _regroup(a):
    return a.reshape(NBLK, 8, 16).transpose(1, 0, 2)


def kernel(tensor_smax):
    xt = tensor_smax.T
    m, loc, l, r, first, last = _shard_scan_tc(xt)
    out2d = _merge_sc(*[_regroup(a) for a in (m, loc, l, r, first, last)])
    return out2d[:, :RPW].reshape(-1)


# speedup vs baseline: 2.5814x; 1.1627x over previous
import jax
import jax.numpy as jnp
from jax import lax
from jax.experimental import pallas as pl
from jax.experimental.pallas import tpu as pltpu
from jax.experimental.pallas import tpu_sc as plsc

R = 128
C = 100000
BLKR = 4000
NBLK = C // BLKR

NC = 2
NS = 16
NW = NC * NS
RPW = R // NW


def _shard_body(x_ref, m_ref, loc_ref, l_ref, r_ref, first_ref, last_ref):
    k = pl.program_id(0)
    sub = lax.broadcasted_iota(jnp.int32, (8, R), 0)
    sub0 = sub == 0

    def body(i, st):
        acc_m, acc_row, lacc, racc, prev_rotp, rowvec = st
        v = x_ref[pl.ds(8 * i, 8), :]
        rotp = pltpu.roll(v, 1, 0)
        rotm = pltpu.roll(v, 7, 0)
        base = k * BLKR + 8 * i
        racc = jnp.where(acc_row == base - 1, rotm, racc)
        upd = v > acc_m
        acc_m = jnp.where(upd, v, acc_m)
        acc_row = jnp.where(upd, rowvec, acc_row)
        lacc = jnp.where(upd, jnp.where(sub0, prev_rotp, rotp), lacc)
        racc = jnp.where(upd, rotm, racc)
        return acc_m, acc_row, lacc, racc, rotp, rowvec + 8

    init = (
        jnp.full((8, R), -jnp.inf, jnp.float32),
        jnp.full((8, R), -2, jnp.int32),
        jnp.zeros((8, R), jnp.float32),
        jnp.zeros((8, R), jnp.float32),
        jnp.zeros((8, R), jnp.float32),
        sub + k * BLKR,
    )
    acc_m, acc_row, lacc, racc, _, _ = lax.fori_loop(
        0, BLKR // 8, body, init, unroll=10)

    m = jnp.max(acc_m, axis=0, keepdims=True)
    loc = jnp.min(jnp.where(acc_m == m, acc_row, jnp.int32(2**30)),
                  axis=0, keepdims=True)
    win = acc_row == loc
    left = jnp.sum(jnp.where(win, lacc, 0.0), axis=0, keepdims=True)
    right = jnp.sum(jnp.where(win, racc, 0.0), axis=0, keepdims=True)
    right = jnp.where(loc == (k + 1) * BLKR - 1, 0.0, right)
    m_ref[...] = m.reshape(1, 1, R)
    loc_ref[...] = loc.reshape(1, 1, R)
    l_ref[...] = left.reshape(1, 1, R)
    r_ref[...] = right.reshape(1, 1, R)
    first_ref[...] = x_ref[0:1, :].reshape(1, 1, R)
    last_ref[...] = x_ref[BLKR - 1:BLKR, :].reshape(1, 1, R)


def _shard_scan_tc(xt):
    o = pl.BlockSpec((1, 1, R), lambda k: (k, 0, 0))
    sf = jax.ShapeDtypeStruct((NBLK, 1, R), jnp.float32)
    si = jax.ShapeDtypeStruct((NBLK, 1, R), jnp.int32)
    return pl.pallas_call(
        _shard_body,
        grid=(NBLK,),
        in_specs=[pl.BlockSpec((BLKR, R), lambda k: (k, 0))],
        out_specs=[o, o, o, o, o, o],
        out_shape=[sf, si, sf, sf, sf, sf],
    )(xt)


def _sc_body(m_hbm, loc_hbm, l_hbm, r_hbm, first_hbm, last_hbm, out_hbm,
             m_v, loc_v, l_v, r_v, first_v, last_v, out_v):
    c = lax.axis_index("c")
    s = lax.axis_index("s")
    w = s * NC + c
    t = w // RPW
    lane = lax.iota(jnp.int32, 16)

    pltpu.sync_copy(m_hbm.at[t], m_v)
    pltpu.sync_copy(loc_hbm.at[t], loc_v)
    pltpu.sync_copy(l_hbm.at[t], l_v)
    pltpu.sync_copy(r_hbm.at[t], r_v)
    pltpu.sync_copy(first_hbm.at[t], first_v)
    pltpu.sync_copy(last_hbm.at[t], last_v)

    def f32(x):
        return x.astype(jnp.float32)

    def eqi(x, const):
        return 1 - jnp.minimum(jnp.abs(x - const), 1)

    M = m_v[0]
    LOC = loc_v[0]
    L = l_v[0]
    RB = r_v[0]
    for sh in range(1, NBLK):
        m_s = m_v[sh]
        loc_s = loc_v[sh]
        fr = f32(eqi(LOC, sh * BLKR - 1))
        RB = first_v[sh] * fr + RB * (1.0 - fr)
        fl = f32(eqi(loc_s, sh * BLKR))
        l_s = last_v[sh - 1] * fl + l_v[sh] * (1.0 - fl)
        uf = jnp.maximum(jnp.sign(m_s - M), 0.0)
        ui = uf.astype(jnp.int32)
        M = m_s * uf + M * (1.0 - uf)
        LOC = loc_s * ui + LOC * (1 - ui)
        L = l_s * uf + L * (1.0 - uf)
        RB = r_v[sh] * uf + RB * (1.0 - uf)

    conf = M + jnp.maximum(L, RB)

    def take(vec, ids):
        dnums = lax.GatherDimensionNumbers(
            offset_dims=(), collapsed_slice_dims=(0,), start_index_map=(0,))
        return lax.gather(vec, ids[:, None], dnums, (1,),
                          mode=lax.GatherScatterMode.PROMISE_IN_BOUNDS)

    out_v[...] = take(conf, 4 * (w % RPW) + (lane & 3))
    pltpu.sync_copy(out_v, out_hbm.at[w])


def _merge_sc(m, loc, l, r, first, last):
    mesh = plsc.VectorSubcoreMesh(core_axis_name="c", subcore_axis_name="s")
    vf = pltpu.VMEM((NBLK, 16), jnp.float32)
    return pl.kernel(
        _sc_body,
        out_type=jax.ShapeDtypeStruct((NW, 16), jnp.float32),
        mesh=mesh,
        scratch_types=[
            vf, pltpu.VMEM((NBLK, 16), jnp.int32), vf, vf, vf, vf,
            pltpu.VMEM((16,), jnp.float32),
        ],
    )(m, loc, l, r, first, last)


def _regroup(a):
    return a.reshape(NBLK, 8, 16).transpose(1, 0, 2)


def kernel(tensor_smax):
    xt = tensor_smax.T
    m, loc, l, r, first, last = _shard_scan_tc(xt)
    out2d = _merge_sc(*[_regroup(a) for a in (m, loc, l, r, first, last)])
    return out2d[:, :RPW].reshape(-1)


# speedup vs baseline: 3.0145x; 1.1678x over previous
import jax
import jax.numpy as jnp
from jax import lax
from jax.experimental import pallas as pl
from jax.experimental.pallas import tpu as pltpu
from jax.experimental.pallas import tpu_sc as plsc

R = 128
C = 100000
BLKR = 4000
NBLK = C // BLKR

NC = 2
NS = 16
NW = NC * NS
RPW = R // NW


def _shard_body(x_ref, m_ref, loc_ref, l_ref, r_ref, first_ref, last_ref):
    k = pl.program_id(0)
    sub = lax.broadcasted_iota(jnp.int32, (8, R), 0)
    sub0 = sub == 0

    def body(i, st):
        acc_m, acc_row, lacc, racc, prev_rotp, rowvec = st
        v = x_ref[pl.ds(8 * i, 8), :]
        rotp = pltpu.roll(v, 1, 0)
        rotm = pltpu.roll(v, 7, 0)
        base = k * BLKR + 8 * i
        racc = jnp.where(acc_row == base - 1, rotm, racc)
        upd = v > acc_m
        acc_m = jnp.where(upd, v, acc_m)
        acc_row = jnp.where(upd, rowvec, acc_row)
        lacc = jnp.where(upd, jnp.where(sub0, prev_rotp, rotp), lacc)
        racc = jnp.where(upd, rotm, racc)
        return acc_m, acc_row, lacc, racc, rotp, rowvec + 8

    init = (
        jnp.full((8, R), -jnp.inf, jnp.float32),
        jnp.full((8, R), -2, jnp.int32),
        jnp.zeros((8, R), jnp.float32),
        jnp.zeros((8, R), jnp.float32),
        jnp.zeros((8, R), jnp.float32),
        sub + k * BLKR,
    )
    acc_m, acc_row, lacc, racc, _, _ = lax.fori_loop(
        0, BLKR // 8, body, init, unroll=10)

    m = jnp.max(acc_m, axis=0, keepdims=True)
    loc = jnp.min(jnp.where(acc_m == m, acc_row, jnp.int32(2**30)),
                  axis=0, keepdims=True)
    win = acc_row == loc
    left = jnp.sum(jnp.where(win, lacc, 0.0), axis=0, keepdims=True)
    right = jnp.sum(jnp.where(win, racc, 0.0), axis=0, keepdims=True)
    right = jnp.where(loc == (k + 1) * BLKR - 1, 0.0, right)
    m_ref[...] = m.reshape(1, 1, R)
    loc_ref[...] = loc.reshape(1, 1, R)
    l_ref[...] = left.reshape(1, 1, R)
    r_ref[...] = right.reshape(1, 1, R)
    first_ref[...] = x_ref[0:1, :].reshape(1, 1, R)
    last_ref[...] = x_ref[BLKR - 1:BLKR, :].reshape(1, 1, R)


def _shard_scan_tc(xt):
    o = pl.BlockSpec((1, 1, R), lambda k: (k, 0, 0))
    sf = jax.ShapeDtypeStruct((NBLK, 1, R), jnp.float32)
    si = jax.ShapeDtypeStruct((NBLK, 1, R), jnp.int32)
    return pl.pallas_call(
        _shard_body,
        grid=(NBLK,),
        in_specs=[pl.BlockSpec((BLKR, R), lambda k: (k, 0))],
        out_specs=[o, o, o, o, o, o],
        out_shape=[sf, si, sf, sf, sf, sf],
    )(xt)


def _sc_body(m_hbm, loc_hbm, l_hbm, r_hbm, first_hbm, last_hbm, out_hbm,
             m_v, loc_v, l_v, r_v, first_v, last_v, out_v,
             s0, s1, s2, s3, s4, s5):
    c = lax.axis_index("c")
    s = lax.axis_index("s")
    w = s * NC + c

    @pl.when(w < 8)
    def _work():
        cps = [
            pltpu.async_copy(m_hbm, m_v, s0),
            pltpu.async_copy(loc_hbm, loc_v, s1),
            pltpu.async_copy(l_hbm, l_v, s2),
            pltpu.async_copy(r_hbm, r_v, s3),
            pltpu.async_copy(first_hbm, first_v, s4),
            pltpu.async_copy(last_hbm, last_v, s5),
        ]
        for cp in cps:
            cp.wait()

        def f32(x):
            return x.astype(jnp.float32)

        def eqi(x, const):
            return 1 - jnp.minimum(jnp.abs(x - const), 1)

        for g in range(8):
            @pl.when(w == g)
            def _group(g=g):
                ds = pl.ds(16 * g, 16)
                M = m_v[0, 0, ds]
                LOC = loc_v[0, 0, ds]
                L = l_v[0, 0, ds]
                RB = r_v[0, 0, ds]
                for sh in range(1, NBLK):
                    m_s = m_v[sh, 0, ds]
                    loc_s = loc_v[sh, 0, ds]
                    fr = f32(eqi(LOC, sh * BLKR - 1))
                    RB = first_v[sh, 0, ds] * fr + RB * (1.0 - fr)
                    fl = f32(eqi(loc_s, sh * BLKR))
                    l_s = last_v[sh - 1, 0, ds] * fl + l_v[sh, 0, ds] * (1.0 - fl)
                    uf = jnp.maximum(jnp.sign(m_s - M), 0.0)
                    ui = uf.astype(jnp.int32)
                    M = m_s * uf + M * (1.0 - uf)
                    LOC = loc_s * ui + LOC * (1 - ui)
                    L = l_s * uf + L * (1.0 - uf)
                    RB = r_v[sh, 0, ds] * uf + RB * (1.0 - uf)

                out_v[...] = M + jnp.maximum(L, RB)
                pltpu.sync_copy(out_v, out_hbm.at[pl.ds(16 * g, 16)])


def _merge_sc(m, loc, l, r, first, last):
    mesh = plsc.VectorSubcoreMesh(core_axis_name="c", subcore_axis_name="s")
    vf = pltpu.VMEM((NBLK, 1, R), jnp.float32)
    return pl.kernel(
        _sc_body,
        out_type=jax.ShapeDtypeStruct((R,), jnp.float32),
        mesh=mesh,
        scratch_types=[
            vf, pltpu.VMEM((NBLK, 1, R), jnp.int32), vf, vf, vf, vf,
            pltpu.VMEM((16,), jnp.float32),
            pltpu.SemaphoreType.DMA, pltpu.SemaphoreType.DMA,
            pltpu.SemaphoreType.DMA, pltpu.SemaphoreType.DMA,
            pltpu.SemaphoreType.DMA, pltpu.SemaphoreType.DMA,
        ],
    )(m, loc, l, r, first, last)


def kernel(tensor_smax):
    xt = tensor_smax.T
    m, loc, l, r, first, last = _shard_scan_tc(xt)
    return _merge_sc(m, loc, l, r, first, last)


# speedup vs baseline: 3.6339x; 1.2055x over previous
import jax
import jax.numpy as jnp
from jax import lax
from jax.experimental import pallas as pl
from jax.experimental.pallas import tpu as pltpu
from jax.experimental.pallas import tpu_sc as plsc

R = 128
C = 100000
BLKR = 10000
NBLK = C // BLKR

NC = 2
NS = 16
NW = NC * NS
RPW = R // NW


def _shard_body(x_ref, m_ref, loc_ref, l_ref, r_ref, first_ref, last_ref):
    k = pl.program_id(0)
    sub = lax.broadcasted_iota(jnp.int32, (8, R), 0)
    sub0 = sub == 0

    def body(i, st):
        acc_m, acc_row, lacc, racc, prev_rotp, rowvec = st
        v = x_ref[pl.ds(8 * i, 8), :]
        rotp = pltpu.roll(v, 1, 0)
        rotm = pltpu.roll(v, 7, 0)
        base = k * BLKR + 8 * i
        racc = jnp.where(acc_row == base - 1, rotm, racc)
        upd = v > acc_m
        acc_m = jnp.where(upd, v, acc_m)
        acc_row = jnp.where(upd, rowvec, acc_row)
        lacc = jnp.where(upd, jnp.where(sub0, prev_rotp, rotp), lacc)
        racc = jnp.where(upd, rotm, racc)
        return acc_m, acc_row, lacc, racc, rotp, rowvec + 8

    init = (
        jnp.full((8, R), -jnp.inf, jnp.float32),
        jnp.full((8, R), -2, jnp.int32),
        jnp.zeros((8, R), jnp.float32),
        jnp.zeros((8, R), jnp.float32),
        jnp.zeros((8, R), jnp.float32),
        sub + k * BLKR,
    )
    acc_m, acc_row, lacc, racc, _, _ = lax.fori_loop(
        0, BLKR // 8, body, init, unroll=10)

    m = jnp.max(acc_m, axis=0, keepdims=True)
    loc = jnp.min(jnp.where(acc_m == m, acc_row, jnp.int32(2**30)),
                  axis=0, keepdims=True)
    win = acc_row == loc
    left = jnp.sum(jnp.where(win, lacc, 0.0), axis=0, keepdims=True)
    right = jnp.sum(jnp.where(win, racc, 0.0), axis=0, keepdims=True)
    right = jnp.where(loc == (k + 1) * BLKR - 1, 0.0, right)
    m_ref[...] = m.reshape(1, 1, R)
    loc_ref[...] = loc.reshape(1, 1, R)
    l_ref[...] = left.reshape(1, 1, R)
    r_ref[...] = right.reshape(1, 1, R)
    first_ref[...] = x_ref[0:1, :].reshape(1, 1, R)
    last_ref[...] = x_ref[BLKR - 1:BLKR, :].reshape(1, 1, R)


def _shard_scan_tc(xt):
    o = pl.BlockSpec((1, 1, R), lambda k: (k, 0, 0))
    sf = jax.ShapeDtypeStruct((NBLK, 1, R), jnp.float32)
    si = jax.ShapeDtypeStruct((NBLK, 1, R), jnp.int32)
    return pl.pallas_call(
        _shard_body,
        grid=(NBLK,),
        in_specs=[pl.BlockSpec((BLKR, R), lambda k: (k, 0))],
        out_specs=[o, o, o, o, o, o],
        out_shape=[sf, si, sf, sf, sf, sf],
    )(xt)


def _sc_body(m_hbm, loc_hbm, l_hbm, r_hbm, first_hbm, last_hbm, out_hbm,
             m_v, loc_v, l_v, r_v, first_v, last_v, out_v,
             s0, s1, s2, s3, s4, s5):
    c = lax.axis_index("c")
    s = lax.axis_index("s")
    w = s * NC + c

    @pl.when(w < 8)
    def _work():
        cps = [
            pltpu.async_copy(m_hbm, m_v, s0),
            pltpu.async_copy(loc_hbm, loc_v, s1),
            pltpu.async_copy(l_hbm, l_v, s2),
            pltpu.async_copy(r_hbm, r_v, s3),
            pltpu.async_copy(first_hbm, first_v, s4),
            pltpu.async_copy(last_hbm, last_v, s5),
        ]
        for cp in cps:
            cp.wait()

        def f32(x):
            return x.astype(jnp.float32)

        def eqi(x, const):
            return 1 - jnp.minimum(jnp.abs(x - const), 1)

        for g in range(8):
            @pl.when(w == g)
            def _group(g=g):
                ds = pl.ds(16 * g, 16)
                M = m_v[0, 0, ds]
                LOC = loc_v[0, 0, ds]
                L = l_v[0, 0, ds]
                RB = r_v[0, 0, ds]
                for sh in range(1, NBLK):
                    m_s = m_v[sh, 0, ds]
                    loc_s = loc_v[sh, 0, ds]
                    fr = f32(eqi(LOC, sh * BLKR - 1))
                    RB = first_v[sh, 0, ds] * fr + RB * (1.0 - fr)
                    fl = f32(eqi(loc_s, sh * BLKR))
                    l_s = last_v[sh - 1, 0, ds] * fl + l_v[sh, 0, ds] * (1.0 - fl)
                    uf = jnp.maximum(jnp.sign(m_s - M), 0.0)
                    ui = uf.astype(jnp.int32)
                    M = m_s * uf + M * (1.0 - uf)
                    LOC = loc_s * ui + LOC * (1 - ui)
                    L = l_s * uf + L * (1.0 - uf)
                    RB = r_v[sh, 0, ds] * uf + RB * (1.0 - uf)

                out_v[...] = M + jnp.maximum(L, RB)
                pltpu.sync_copy(out_v, out_hbm.at[pl.ds(16 * g, 16)])


def _merge_sc(m, loc, l, r, first, last):
    mesh = plsc.VectorSubcoreMesh(core_axis_name="c", subcore_axis_name="s")
    vf = pltpu.VMEM((NBLK, 1, R), jnp.float32)
    return pl.kernel(
        _sc_body,
        out_type=jax.ShapeDtypeStruct((R,), jnp.float32),
        mesh=mesh,
        scratch_types=[
            vf, pltpu.VMEM((NBLK, 1, R), jnp.int32), vf, vf, vf, vf,
            pltpu.VMEM((16,), jnp.float32),
            pltpu.SemaphoreType.DMA, pltpu.SemaphoreType.DMA,
            pltpu.SemaphoreType.DMA, pltpu.SemaphoreType.DMA,
            pltpu.SemaphoreType.DMA, pltpu.SemaphoreType.DMA,
        ],
    )(m, loc, l, r, first, last)


def kernel(tensor_smax):
    xt = tensor_smax.T
    m, loc, l, r, first, last = _shard_scan_tc(xt)
    return _merge_sc(m, loc, l, r, first, last)
